# baseline (speedup 1.0000x reference)
import jax
import jax.numpy as jnp
from jax.experimental import pallas as pl

B, S, D = 4, 2048, 1024
D_BLK = 512
ND = D // D_BLK


def _add_body(x_ref, pe_ref, out_ref):
    out_ref[...] = x_ref[...] + pe_ref[...]


def kernel(x, pe_table):
    batch, seq_len, d_model = x.shape
    pe = pe_table[:seq_len]
    x2 = x.reshape(batch * seq_len, d_model)
    out = pl.pallas_call(
        _add_body,
        grid=(ND, batch),
        in_specs=[
            pl.BlockSpec((seq_len, D_BLK), lambda d, b: (b, d)),
            pl.BlockSpec((seq_len, D_BLK), lambda d, b: (0, d)),
        ],
        out_specs=pl.BlockSpec((seq_len, D_BLK), lambda d, b: (b, d)),
        out_shape=jax.ShapeDtypeStruct((batch * seq_len, d_model), x.dtype),
    )(x2, pe)
    return out.reshape(batch, seq_len, d_model)


# speedup vs baseline: 1.1115x; 1.1115x over previous
import jax
import jax.numpy as jnp
from jax.experimental import pallas as pl

B, S, D = 4, 2048, 1024


def _add_body(x_ref, pe_ref, out_ref):
    out_ref[...] = x_ref[...] + pe_ref[...]


def kernel(x, pe_table):
    batch, seq_len, d_model = x.shape
    pe = pe_table[:seq_len]
    x2 = x.reshape(batch * seq_len, d_model)
    out = pl.pallas_call(
        _add_body,
        grid=(batch,),
        in_specs=[
            pl.BlockSpec((seq_len, d_model), lambda b: (b, 0)),
            pl.BlockSpec((seq_len, d_model), lambda b: (0, 0)),
        ],
        out_specs=pl.BlockSpec((seq_len, d_model), lambda b: (b, 0)),
        out_shape=jax.ShapeDtypeStruct((batch * seq_len, d_model), x.dtype),
    )(x2, pe)
    return out.reshape(batch, seq_len, d_model)
